# baseline (speedup 1.0000x reference)
import functools

import jax
import jax.numpy as jnp
from jax import lax
from jax.experimental import pallas as pl
from jax.experimental.pallas import tpu as pltpu
from jax.experimental.pallas import tpu_sc as plsc

_VOCAB = 1000000
_D = 64
_B = 16384
_H = 50

_NC = 2
_NS = 16
_NW = _NC * _NS

_SPW = _B // _NW
_CHS = 16
_NCH = _SPW // _CHS


@jax.jit
def _gather(idx, table):
    mesh = plsc.VectorSubcoreMesh(
        core_axis_name="c", subcore_axis_name="s",
        num_cores=_NC, num_subcores=_NS)

    @functools.partial(
        pl.kernel,
        out_type=jax.ShapeDtypeStruct((_B, _H, _D), jnp.float32),
        mesh=mesh,
        scratch_types=[
            pltpu.VMEM((_CHS * _H,), jnp.int32),
            pltpu.VMEM((_CHS * _H,), jnp.int32),
            pltpu.VMEM((_CHS * _H, _D), jnp.float32),
            pltpu.VMEM((_CHS * _H, _D), jnp.float32),
            pltpu.SemaphoreType.DMA,
            pltpu.SemaphoreType.DMA,
            pltpu.SemaphoreType.DMA,
            pltpu.SemaphoreType.DMA,
            pltpu.SemaphoreType.DMA,
            pltpu.SemaphoreType.DMA,
        ],
        compiler_params=pltpu.CompilerParams(use_tc_tiling_on_sc=False),
    )
    def k(idx_hbm, table_hbm, out_hbm, idx0, idx1, rows0, rows1,
          g0, g1, s0, s1, i0, i1):
        wid = lax.axis_index("s") * _NC + lax.axis_index("c")
        base = wid * _SPW
        fbase = base * _H
        idxb = (idx0, idx1)
        rows = (rows0, rows1)
        gsem = (g0, g1)
        ssem = (s0, s1)
        isem = (i0, i1)

        def start_idx(i, b):
            pltpu.async_copy(idx_hbm.at[pl.ds(fbase + i * _CHS * _H, _CHS * _H)],
                             idxb[b], isem[b])

        def wait_idx(b):
            pltpu.make_async_copy(idx_hbm.at[pl.ds(fbase, _CHS * _H)],
                                  idxb[b], isem[b]).wait()

        def start_gather(b):
            pltpu.async_copy(table_hbm.at[idxb[b]], rows[b], gsem[b])

        def wait_gather(b):
            pltpu.make_async_copy(table_hbm.at[idxb[b]],
                                  rows[b], gsem[b]).wait()

        def start_scatter(i, b):
            s0_ = base + i * _CHS
            for j in range(_CHS):
                pltpu.async_copy(rows[b].at[pl.ds(j * _H, _H)],
                                 out_hbm.at[s0_ + j], ssem[b])

        def wait_scatter(b):
            for j in range(_CHS):
                pltpu.make_async_copy(rows[b].at[pl.ds(0, _H)],
                                      out_hbm.at[base], ssem[b]).wait()

        def step(i, b):
            ob = 1 - b
            wait_gather(b)
            start_scatter(i, b)

            @pl.when(i + 1 < _NCH)
            def _():
                @pl.when(i > 0)
                def _():
                    wait_scatter(ob)
                wait_idx(ob)
                start_gather(ob)

            @pl.when(i + 2 < _NCH)
            def _():
                start_idx(i + 2, b)

        start_idx(0, 0)
        wait_idx(0)
        start_gather(0)
        start_idx(1, 1)

        def body(j, carry):
            step(2 * j, 0)
            step(2 * j + 1, 1)
            return carry

        lax.fori_loop(0, _NCH // 2, body, 0)
        wait_scatter(0)
        wait_scatter(1)

    return k(idx, table)


def kernel(x, table):
    return _gather(x.reshape(-1).astype(jnp.int32), table)


# speedup vs baseline: 1.0051x; 1.0051x over previous
import functools

import jax
import jax.numpy as jnp
from jax import lax
from jax.experimental import pallas as pl
from jax.experimental.pallas import tpu as pltpu
from jax.experimental.pallas import tpu_sc as plsc

_VOCAB = 1000000
_D = 64
_B = 16384
_H = 50

_NC = 2
_NS = 16
_NW = _NC * _NS

_SPW = _B // _NW
_CHS = 16
_NCH = _SPW // _CHS


@jax.jit
def _gather(idx, table):
    mesh = plsc.VectorSubcoreMesh(
        core_axis_name="c", subcore_axis_name="s",
        num_cores=_NC, num_subcores=_NS)

    @functools.partial(
        pl.kernel,
        out_type=jax.ShapeDtypeStruct((_B, _H, _D), jnp.float32),
        mesh=mesh,
        scratch_types=[
            pltpu.VMEM((_CHS * _H,), jnp.int32),
            pltpu.VMEM((_CHS * _H,), jnp.int32),
            pltpu.VMEM((_CHS * _H, _D), jnp.float32),
            pltpu.VMEM((_CHS * _H, _D), jnp.float32),
            pltpu.SemaphoreType.DMA,
            pltpu.SemaphoreType.DMA,
            pltpu.SemaphoreType.DMA,
            pltpu.SemaphoreType.DMA,
            pltpu.SemaphoreType.DMA,
            pltpu.SemaphoreType.DMA,
        ],
        compiler_params=pltpu.CompilerParams(use_tc_tiling_on_sc=False),
    )
    def k(idx_hbm, table_hbm, out_hbm, idx0, idx1, rows0, rows1,
          g0, g1, s0, s1, i0, i1):
        wid = lax.axis_index("s") * _NC + lax.axis_index("c")
        base = wid * _SPW
        cbase = wid * _NCH
        idxb = (idx0, idx1)
        rows = (rows0, rows1)
        gsem = (g0, g1)
        ssem = (s0, s1)
        isem = (i0, i1)

        def start_idx(i, b):
            pltpu.async_copy(idx_hbm.at[cbase + i], idxb[b], isem[b])

        def wait_idx(b):
            pltpu.make_async_copy(idx_hbm.at[cbase], idxb[b], isem[b]).wait()

        def start_gather(b):
            pltpu.async_copy(table_hbm.at[idxb[b]], rows[b], gsem[b])

        def wait_gather(b):
            pltpu.make_async_copy(table_hbm.at[idxb[b]],
                                  rows[b], gsem[b]).wait()

        def start_scatter(i, b):
            s0_ = base + i * _CHS
            for j in range(_CHS):
                pltpu.async_copy(rows[b].at[pl.ds(j * _H, _H)],
                                 out_hbm.at[s0_ + j], ssem[b])

        def wait_scatter(b):
            for j in range(_CHS):
                pltpu.make_async_copy(rows[b].at[pl.ds(0, _H)],
                                      out_hbm.at[base], ssem[b]).wait()

        def step(i, b):
            ob = 1 - b
            wait_gather(b)
            start_scatter(i, b)

            @pl.when(i + 1 < _NCH)
            def _():
                @pl.when(i > 0)
                def _():
                    wait_scatter(ob)
                wait_idx(ob)
                start_gather(ob)

            @pl.when(i + 2 < _NCH)
            def _():
                start_idx(i + 2, b)

        start_idx(0, 0)
        wait_idx(0)
        start_gather(0)
        start_idx(1, 1)

        def body(j, carry):
            step(2 * j, 0)
            step(2 * j + 1, 1)
            return carry

        lax.fori_loop(0, _NCH // 2, body, 0)
        wait_scatter(0)
        wait_scatter(1)

    return k(idx, table)


def kernel(x, table):
    idx = x.reshape(_NW * _NCH, _CHS * _H).astype(jnp.int32)
    return _gather(idx, table)
